# baseline (speedup 1.0000x reference)
import jax
import jax.numpy as jnp
from jax.experimental import pallas as pl
from jax.experimental.pallas import tpu as pltpu

_BLOCK_ROWS = 256


def _gap_sq_kernel(q_ref, in_ref, tgt_ref, out_ref):
    qcol = q_ref[...].reshape(_BLOCK_ROWS, 1)
    i = in_ref[...]
    t = tgt_ref[...]
    diff = i - t
    gap = jnp.where(qcol == 1, t * diff, diff)
    out_ref[...] = gap * gap


def kernel(input_y, target_y, q):
    n, d = input_y.shape
    b = _BLOCK_ROWS
    g = n // b
    return pl.pallas_call(
        _gap_sq_kernel,
        grid=(g,),
        in_specs=[
            pl.BlockSpec((b,), lambda i: (i,)),
            pl.BlockSpec((b, d), lambda i: (i, 0)),
            pl.BlockSpec((b, d), lambda i: (i, 0)),
        ],
        out_specs=pl.BlockSpec((b, d), lambda i: (i, 0)),
        out_shape=jax.ShapeDtypeStruct((n, d), jnp.float32),
        compiler_params=pltpu.CompilerParams(
            dimension_semantics=("arbitrary",),
        ),
    )(q.astype(jnp.int32), input_y, target_y)


# speedup vs baseline: 1.0277x; 1.0277x over previous
import jax
import jax.numpy as jnp
from jax.experimental import pallas as pl
from jax.experimental.pallas import tpu as pltpu

_BLOCK_ROWS = 512


def _gap_sq_kernel(q_ref, in_ref, tgt_ref, out_ref):
    qcol = q_ref[...].reshape(_BLOCK_ROWS, 1)
    i = in_ref[...]
    t = tgt_ref[...]
    diff = i - t
    gap = jnp.where(qcol == 1, t * diff, diff)
    out_ref[...] = gap * gap


def kernel(input_y, target_y, q):
    n, d = input_y.shape
    b = _BLOCK_ROWS
    g = n // b
    return pl.pallas_call(
        _gap_sq_kernel,
        grid=(g,),
        in_specs=[
            pl.BlockSpec((b,), lambda i: (i,)),
            pl.BlockSpec((b, d), lambda i: (i, 0)),
            pl.BlockSpec((b, d), lambda i: (i, 0)),
        ],
        out_specs=pl.BlockSpec((b, d), lambda i: (i, 0)),
        out_shape=jax.ShapeDtypeStruct((n, d), jnp.float32),
        compiler_params=pltpu.CompilerParams(
            dimension_semantics=("arbitrary",),
        ),
    )(q.astype(jnp.int32), input_y, target_y)


# speedup vs baseline: 1.0303x; 1.0026x over previous
import jax
import jax.numpy as jnp
from jax.experimental import pallas as pl
from jax.experimental.pallas import tpu as pltpu

_BLOCK_ROWS = 512


def _gap_sq_kernel(q_ref, in_ref, tgt_ref, out_ref):
    qcol = q_ref[...].reshape(_BLOCK_ROWS, 1)
    i = in_ref[...]
    t = tgt_ref[...]
    diff = i - t
    gap = jnp.where(qcol == 1, t * diff, diff)
    out_ref[...] = gap * gap


def kernel(input_y, target_y, q):
    n, d = input_y.shape
    b = _BLOCK_ROWS
    g = n // b
    return pl.pallas_call(
        _gap_sq_kernel,
        grid=(g,),
        in_specs=[
            pl.BlockSpec((b,), lambda i: (i,)),
            pl.BlockSpec((b, d), lambda i: (i, 0)),
            pl.BlockSpec((b, d), lambda i: (i, 0)),
        ],
        out_specs=pl.BlockSpec((b, d), lambda i: (i, 0)),
        out_shape=jax.ShapeDtypeStruct((n, d), jnp.float32),
        compiler_params=pltpu.CompilerParams(
            dimension_semantics=("parallel",),
        ),
    )(q.astype(jnp.int32), input_y, target_y)
